# baseline (speedup 1.0000x reference)
import functools

import jax
import jax.numpy as jnp
from jax import lax
from jax.experimental import pallas as pl
from jax.experimental.pallas import tpu as pltpu
from jax.experimental.pallas import tpu_sc as plsc

_N = 10000
_E = 320000
_R = 8
_D = 128

_NC = 2
_NS = 16
_NW = _NC * _NS
_CH = 128
_CPW = 80
_EP = _NW * _CPW * _CH
_NROWS = _EP // _CH
_NPAD = 10240
_RPT = _NPAD // _NS
_LANES = 16


_HALF = _CPW // 2


def _sc_gather_scatter_body(table, srcs, etys, dsts, out,
                            idx_v, ety_v, dst_v, row_a, row_b, agg_sh,
                            sem_a, sem_b, ssem_a, ssem_b):
    c = lax.axis_index("c")
    s = lax.axis_index("s")
    w = c * _NS + s
    base = w * _CPW

    def _zrow(j, carry):
        for k in range(_D // _LANES):
            row_a[j, pl.ds(k * _LANES, _LANES)] = jnp.zeros(
                (_LANES,), jnp.float32)
        return carry
    lax.fori_loop(0, _CH, _zrow, 0)
    for i in range(_RPT // _CH):
        pltpu.sync_copy(row_a, agg_sh.at[pl.ds(s * _RPT + i * _CH, _CH)])

    plsc.subcore_barrier()

    def _gstart(j, buf, sem):
        pltpu.make_async_copy(table.at[idx_v.at[j]], buf, sem).start()

    def _gwait(j, buf, sem):
        pltpu.make_async_copy(table.at[idx_v.at[j]], buf, sem).wait()

    for half in range(2):
        hbase = base + half * _HALF
        pltpu.sync_copy(srcs.at[pl.ds(hbase, _HALF)], idx_v)
        pltpu.sync_copy(etys.at[pl.ds(hbase, _HALF)], ety_v)
        pltpu.sync_copy(dsts.at[pl.ds(hbase, _HALF)], dst_v)

        def _idxrow(j, carry):
            for k in range(_CH // _LANES):
                sl = (j, pl.ds(k * _LANES, _LANES))
                idx_v[sl] = ety_v[sl] * _N + idx_v[sl]
            return carry
        lax.fori_loop(0, _HALF, _idxrow, 0)

        def _sstart(j, buf, sem):
            pltpu.async_copy(buf, agg_sh.at[dst_v.at[j]], sem, add=True)

        def _swait(buf, sem):
            pltpu.make_async_copy(buf, agg_sh.at[dst_v.at[0]], sem).wait()

        _gstart(0, row_a, sem_a)
        _gstart(1, row_b, sem_b)

        def _pair(i, carry):
            j0 = 2 * i
            j1 = j0 + 1
            _gwait(j0, row_a, sem_a)
            _sstart(j0, row_a, ssem_a)
            _gwait(j1, row_b, sem_b)
            _sstart(j1, row_b, ssem_b)

            @pl.when(i + 1 < _HALF // 2)
            def _():
                _swait(row_a, ssem_a)
                _gstart(j0 + 2, row_a, sem_a)
                _swait(row_b, ssem_b)
                _gstart(j1 + 2, row_b, sem_b)

            return carry

        lax.fori_loop(0, _HALF // 2, _pair, 0)
        _swait(row_a, ssem_a)
        _swait(row_b, ssem_b)

    plsc.subcore_barrier()

    pltpu.sync_copy(agg_sh.at[pl.ds(s * _RPT, _RPT)],
                    out.at[c, pl.ds(s * _RPT, _RPT)])


_sc_gather_scatter = functools.partial(
    pl.kernel,
    out_type=jax.ShapeDtypeStruct((_NC, _NPAD, _D), jnp.float32),
    mesh=plsc.VectorSubcoreMesh(core_axis_name="c", subcore_axis_name="s"),
    scratch_types=[
        pltpu.VMEM((_HALF, _CH), jnp.int32),
        pltpu.VMEM((_HALF, _CH), jnp.int32),
        pltpu.VMEM((_HALF, _CH), jnp.int32),
        pltpu.VMEM((_CH, _D), jnp.float32),
        pltpu.VMEM((_CH, _D), jnp.float32),
        pltpu.VMEM_SHARED((_NPAD, _D), jnp.float32),
        pltpu.SemaphoreType.DMA,
        pltpu.SemaphoreType.DMA,
        pltpu.SemaphoreType.DMA,
        pltpu.SemaphoreType.DMA,
    ],
)(_sc_gather_scatter_body)


_BN = 400
_NB = _N // _BN


def _mm_body(x_ref, w_ref, o_ref):
    o_ref[...] = jnp.dot(x_ref[...], w_ref[0],
                         preferred_element_type=jnp.float32)


def _mm_all(x, wall):
    return pl.pallas_call(
        _mm_body,
        grid=(_R + 1,),
        in_specs=[
            pl.BlockSpec((_N, _D), lambda r: (0, 0)),
            pl.BlockSpec((1, _D, _D), lambda r: (r, 0, 0)),
        ],
        out_specs=pl.BlockSpec((_N, _D), lambda r: (r, 0)),
        out_shape=jax.ShapeDtypeStruct(((_R + 1) * _N, _D), jnp.float32),
    )(x, wall)


def _epi_body(a_ref, xw_ref, b_ref, o_ref, *, act):
    h = a_ref[0] + a_ref[1] + xw_ref[...] + b_ref[...]
    o_ref[...] = jnp.maximum(h, 0.0) if act else h


def _epilogue(aggs, xwall, b, act):
    return pl.pallas_call(
        functools.partial(_epi_body, act=act),
        grid=(_NB,),
        in_specs=[
            pl.BlockSpec((_NC, _BN, _D), lambda j: (0, j, 0)),
            pl.BlockSpec((_BN, _D), lambda j: (_R * _NB + j, 0)),
            pl.BlockSpec((1, _D), lambda j: (0, 0)),
        ],
        out_specs=pl.BlockSpec((_BN, _D), lambda j: (j, 0)),
        out_shape=jax.ShapeDtypeStruct((_N, _D), jnp.float32),
    )(aggs, xwall, b.reshape(1, _D))


def _layer(x, wall, b, srcs, etys, dsts, act):
    table = _mm_all(x, wall)
    aggs = _sc_gather_scatter(table, srcs, etys, dsts)
    return _epilogue(aggs, table, b, act)


def kernel(feats, edge_index, etype, W1, Wloop1, b1, W2, Wloop2, b2):
    src = edge_index[0]
    dst = edge_index[1]
    pad = _EP - _E
    zpad = jnp.zeros((pad,), jnp.int32)
    spad = jnp.arange(pad, dtype=jnp.int32) % _N
    dpad = _N + (jnp.arange(pad, dtype=jnp.int32) % (_NPAD - _N))
    srcs = jnp.concatenate([src, spad]).reshape(_NROWS, _CH)
    etys = jnp.concatenate([etype, zpad]).reshape(_NROWS, _CH)
    dsts = jnp.concatenate([dst, dpad]).reshape(_NROWS, _CH)
    wall1 = jnp.concatenate([W1, Wloop1[None]], axis=0)
    wall2 = jnp.concatenate([W2, Wloop2[None]], axis=0)
    h = _layer(feats, wall1, b1, srcs, etys, dsts, act=True)
    h = _layer(h, wall2, b2, srcs, etys, dsts, act=False)
    return h


# speedup vs baseline: 1.0804x; 1.0804x over previous
import functools

import jax
import jax.numpy as jnp
from jax import lax
from jax.experimental import pallas as pl
from jax.experimental.pallas import tpu as pltpu
from jax.experimental.pallas import tpu_sc as plsc

_N = 10000
_E = 320000
_R = 8
_D = 128

_NC = 2
_NS = 16
_NW = _NC * _NS
_CH = 128
_CPW = 80
_EP = _NW * _CPW * _CH
_NROWS = _EP // _CH
_NPAD = 10240
_RPT = _NPAD // _NS
_LANES = 16


_HALF = _CPW // 2


def _sc_gather_scatter_body(table, srcs, etys, dsts, out,
                            idx_v, ety_v, dst_v, row_a, row_b, agg_sh,
                            sem_a, sem_b):
    c = lax.axis_index("c")
    s = lax.axis_index("s")
    w = c * _NS + s
    base = w * _CPW

    def _zrow(j, carry):
        for k in range(_D // _LANES):
            row_a[j, pl.ds(k * _LANES, _LANES)] = jnp.zeros(
                (_LANES,), jnp.float32)
        return carry
    lax.fori_loop(0, _CH, _zrow, 0)
    for i in range(_RPT // _CH):
        pltpu.sync_copy(row_a, agg_sh.at[pl.ds(s * _RPT + i * _CH, _CH)])

    plsc.subcore_barrier()

    def _gstart(j, buf, sem):
        pltpu.make_async_copy(table.at[idx_v.at[j]], buf, sem).start()

    def _gwait(j, buf, sem):
        pltpu.make_async_copy(table.at[idx_v.at[j]], buf, sem).wait()

    for half in range(2):
        hbase = base + half * _HALF
        pltpu.sync_copy(srcs.at[pl.ds(hbase, _HALF)], idx_v)
        pltpu.sync_copy(etys.at[pl.ds(hbase, _HALF)], ety_v)
        pltpu.sync_copy(dsts.at[pl.ds(hbase, _HALF)], dst_v)

        def _idxrow(j, carry):
            for k in range(_CH // _LANES):
                sl = (j, pl.ds(k * _LANES, _LANES))
                idx_v[sl] = ety_v[sl] * _N + idx_v[sl]
            return carry
        lax.fori_loop(0, _HALF, _idxrow, 0)

        _gstart(0, row_a, sem_a)

        def _pair(i, carry):
            j0 = 2 * i
            j1 = j0 + 1
            _gwait(j0, row_a, sem_a)
            _gstart(j1, row_b, sem_b)
            pltpu.sync_copy(row_a, agg_sh.at[dst_v.at[j0]], add=True)
            _gwait(j1, row_b, sem_b)

            @pl.when(j0 + 2 < _HALF)
            def _():
                _gstart(j0 + 2, row_a, sem_a)

            pltpu.sync_copy(row_b, agg_sh.at[dst_v.at[j1]], add=True)
            return carry

        lax.fori_loop(0, _HALF // 2, _pair, 0)

    plsc.subcore_barrier()

    pltpu.sync_copy(agg_sh.at[pl.ds(s * _RPT, _RPT)],
                    out.at[c, pl.ds(s * _RPT, _RPT)])


_sc_gather_scatter = functools.partial(
    pl.kernel,
    out_type=jax.ShapeDtypeStruct((_NC, _NPAD, _D), jnp.float32),
    mesh=plsc.VectorSubcoreMesh(core_axis_name="c", subcore_axis_name="s"),
    scratch_types=[
        pltpu.VMEM((_HALF, _CH), jnp.int32),
        pltpu.VMEM((_HALF, _CH), jnp.int32),
        pltpu.VMEM((_HALF, _CH), jnp.int32),
        pltpu.VMEM((_CH, _D), jnp.float32),
        pltpu.VMEM((_CH, _D), jnp.float32),
        pltpu.VMEM_SHARED((_NPAD, _D), jnp.float32),
        pltpu.SemaphoreType.DMA,
        pltpu.SemaphoreType.DMA,
    ],
)(_sc_gather_scatter_body)


_BN = 400
_NB = _N // _BN


def _mm_body(x_ref, w_ref, o_ref):
    o_ref[...] = jnp.dot(x_ref[...], w_ref[0],
                         preferred_element_type=jnp.float32)


def _mm_all(x, wall):
    return pl.pallas_call(
        _mm_body,
        grid=(_R + 1,),
        in_specs=[
            pl.BlockSpec((_N, _D), lambda r: (0, 0)),
            pl.BlockSpec((1, _D, _D), lambda r: (r, 0, 0)),
        ],
        out_specs=pl.BlockSpec((_N, _D), lambda r: (r, 0)),
        out_shape=jax.ShapeDtypeStruct(((_R + 1) * _N, _D), jnp.float32),
    )(x, wall)


def _epi_body(a_ref, xw_ref, b_ref, o_ref, *, act):
    h = a_ref[0] + a_ref[1] + xw_ref[...] + b_ref[...]
    o_ref[...] = jnp.maximum(h, 0.0) if act else h


def _epilogue(aggs, xwall, b, act):
    return pl.pallas_call(
        functools.partial(_epi_body, act=act),
        grid=(_NB,),
        in_specs=[
            pl.BlockSpec((_NC, _BN, _D), lambda j: (0, j, 0)),
            pl.BlockSpec((_BN, _D), lambda j: (_R * _NB + j, 0)),
            pl.BlockSpec((1, _D), lambda j: (0, 0)),
        ],
        out_specs=pl.BlockSpec((_BN, _D), lambda j: (j, 0)),
        out_shape=jax.ShapeDtypeStruct((_N, _D), jnp.float32),
    )(aggs, xwall, b.reshape(1, _D))


def _layer(x, wall, b, srcs, etys, dsts, act):
    table = _mm_all(x, wall)
    aggs = _sc_gather_scatter(table, srcs, etys, dsts)
    return _epilogue(aggs, table, b, act)


def kernel(feats, edge_index, etype, W1, Wloop1, b1, W2, Wloop2, b2):
    src = edge_index[0]
    dst = edge_index[1]
    pad = _EP - _E
    zpad = jnp.zeros((pad,), jnp.int32)
    spad = jnp.arange(pad, dtype=jnp.int32) % _N
    dpad = _N + (jnp.arange(pad, dtype=jnp.int32) % (_NPAD - _N))
    srcs = jnp.concatenate([src, spad]).reshape(_NROWS, _CH)
    etys = jnp.concatenate([etype, zpad]).reshape(_NROWS, _CH)
    dsts = jnp.concatenate([dst, dpad]).reshape(_NROWS, _CH)
    wall1 = jnp.concatenate([W1, Wloop1[None]], axis=0)
    wall2 = jnp.concatenate([W2, Wloop2[None]], axis=0)
    h = _layer(feats, wall1, b1, srcs, etys, dsts, act=True)
    h = _layer(h, wall2, b2, srcs, etys, dsts, act=False)
    return h
